# Initial kernel scaffold; baseline (speedup 1.0000x reference)
import jax
import jax.numpy as jnp
from jax import lax
from jax.experimental import pallas as pl
from jax.experimental.pallas import tpu as pltpu
from jax.experimental.pallas import tpu_sc as plsc

NN = 100000
NE = 1600000
BT = 32
HB = 16
K = 6
NT = 4

NPAD = 100352
RPT = NPAD // 16
SCH = 896
NCK = NE // 128
CPT = (NCK + 15) // 16

_mesh = plsc.VectorSubcoreMesh(core_axis_name="c", subcore_axis_name="s")


def _deg_body(eidx, deg, dacc, ibuf, obuf, zbuf):
  c = lax.axis_index("c")
  s = lax.axis_index("s")

  @pl.loop(0, SCH // 16)
  def _(i):
    zbuf[pl.ds(i * 16, 16)] = jnp.zeros((16,), jnp.float32)

  @pl.loop(0, 8)
  def _(i):
    obuf[pl.ds(i * 16, 16)] = jnp.ones((16,), jnp.float32)

  @pl.loop(0, RPT // SCH)
  def _(i):
    pltpu.sync_copy(zbuf, dacc.at[pl.ds(s * RPT + i * SCH, SCH)])

  plsc.subcore_barrier()

  lo = s * CPT
  hi = jnp.minimum(lo + CPT, NCK)

  @pl.loop(lo, hi)
  def _(j):
    pltpu.sync_copy(eidx.at[c, j], ibuf)
    pltpu.sync_copy(obuf, dacc.at[ibuf], add=True)

  plsc.subcore_barrier()

  @pl.loop(0, RPT // SCH)
  def _(i):
    r0 = s * RPT + i * SCH
    pltpu.sync_copy(dacc.at[pl.ds(r0, SCH)], deg.at[c, pl.ds(r0, SCH)])


_deg_call = pl.kernel(
    _deg_body,
    out_type=jax.ShapeDtypeStruct((2, NPAD), jnp.float32),
    mesh=_mesh,
    scratch_types=[
        pltpu.VMEM_SHARED((NPAD,), jnp.float32),
        pltpu.VMEM((128,), jnp.int32),
        pltpu.VMEM((128,), jnp.float32),
        pltpu.VMEM((SCH,), jnp.float32),
    ],
)


_NBB = 6272


def _prep_body(deg_ref, bi_ref, ab_ref, bb_ref, g0_ref, m0_ref):
  i = pl.program_id(0)
  dsr = deg_ref[0, :]
  ddt = deg_ref[1, :]
  a = jnp.where(dsr > 0, lax.rsqrt(dsr), 0.0)
  b = jnp.where(ddt > 0, lax.rsqrt(ddt), 0.0)
  ab_ref[...] = a * b
  bb_ref[...] = b
  ids = i * _NBB + lax.broadcasted_iota(jnp.int32, (_NBB,), 0)
  oh = ids[:, None] == bi_ref[...][None, :]
  m0_ref[...] = oh.astype(jnp.float32)
  g0_ref[...] = jnp.where(oh, a[:, None], 0.0)


def _prep_call(deg, bi):
  return pl.pallas_call(
      _prep_body,
      grid=(NPAD // _NBB,),
      in_specs=[
          pl.BlockSpec((2, _NBB), lambda i: (0, i)),
          pl.BlockSpec((BT,), lambda i: (0,)),
      ],
      out_specs=[
          pl.BlockSpec((_NBB,), lambda i: (i,)),
          pl.BlockSpec((_NBB,), lambda i: (i,)),
          pl.BlockSpec((_NBB, BT), lambda i: (i, 0)),
          pl.BlockSpec((_NBB, BT), lambda i: (i, 0)),
      ],
      out_shape=[
          jax.ShapeDtypeStruct((NPAD,), jnp.float32),
          jax.ShapeDtypeStruct((NPAD,), jnp.float32),
          jax.ShapeDtypeStruct((NPAD, BT), jnp.float32),
          jax.ShapeDtypeStruct((NPAD, BT), jnp.float32),
      ],
  )(deg, bi)


def _prop_body(eidx, ab, bb, g0, mon, gw,
               acc, sidx, didx, rbuf, tbuf, gbuf, mbuf, abuf, bbuf, zbuf):
  c = lax.axis_index("c")
  s = lax.axis_index("s")
  goff = c * NPAD

  @pl.loop(0, SCH)
  def _(n):
    zbuf[n] = jnp.zeros((16,), jnp.float32)

  @pl.loop(0, RPT // SCH)
  def _(i):
    r0 = s * RPT + i * SCH
    pltpu.sync_copy(g0.at[pl.ds(r0, SCH), pl.ds(c * HB, HB)], tbuf)
    pltpu.sync_copy(tbuf, gw.at[pl.ds(goff + r0, SCH)])
    pltpu.sync_copy(zbuf, acc.at[pl.ds(r0, SCH)])

  plsc.subcore_barrier()

  lo = s * CPT
  hi = jnp.minimum(lo + CPT, NCK)

  @pl.loop(0, K)
  def _(it):
    @pl.loop(lo, hi)
    def _(j):
      pltpu.sync_copy(eidx.at[0, j], sidx)
      pltpu.sync_copy(eidx.at[1, j], didx)

      @pl.loop(0, 8)
      def _(q):
        sidx[pl.ds(q * 16, 16)] = sidx[pl.ds(q * 16, 16)] + goff

      pltpu.sync_copy(gw.at[sidx], rbuf)
      pltpu.sync_copy(rbuf, acc.at[didx], add=True)

    plsc.subcore_barrier()

    @pl.loop(0, RPT // SCH)
    def _(i):
      r0 = s * RPT + i * SCH
      pltpu.sync_copy(acc.at[pl.ds(r0, SCH)], tbuf)
      pltpu.sync_copy(ab.at[pl.ds(r0, SCH)], abuf)
      pltpu.sync_copy(bb.at[pl.ds(r0, SCH)], bbuf)
      pltpu.sync_copy(zbuf, acc.at[pl.ds(r0, SCH)])

      @pl.loop(0, SCH)
      def _(n):
        t = tbuf[n]
        gbuf[n] = abuf[n] * t
        mbuf[n] = bbuf[n] * t

      pltpu.sync_copy(gbuf, gw.at[pl.ds(goff + r0, SCH)])
      pltpu.sync_copy(mbuf, mon.at[it, pl.ds(r0, SCH), c])

    plsc.subcore_barrier()


_prop_call = pl.kernel(
    _prop_body,
    out_type=[
        jax.ShapeDtypeStruct((K, NPAD, 2, HB), jnp.float32),
        jax.ShapeDtypeStruct((2 * NPAD, HB), jnp.float32),
    ],
    mesh=_mesh,
    scratch_types=[
        pltpu.VMEM_SHARED((NPAD, HB), jnp.float32),
        pltpu.VMEM((128,), jnp.int32),
        pltpu.VMEM((128,), jnp.int32),
        pltpu.VMEM((128, HB), jnp.float32),
        pltpu.VMEM((SCH, HB), jnp.float32),
        pltpu.VMEM((SCH, HB), jnp.float32),
        pltpu.VMEM((SCH, HB), jnp.float32),
        pltpu.VMEM((SCH,), jnp.float32),
        pltpu.VMEM((SCH,), jnp.float32),
        pltpu.VMEM((SCH, HB), jnp.float32),
    ],
)


_RB = 25


def _tay_body(mon_ref, m0_ref, co_ref, out_ref):
  co = co_ref[...]
  for t in range(NT):
    acc = co[t, 0] * m0_ref[...]
    for k in range(K):
      acc = acc + co[t, k + 1] * mon_ref[k]
    out_ref[t] = acc


def _tay_call(monr, m0r, coeffs):
  return pl.pallas_call(
      _tay_body,
      grid=(3125 // _RB,),
      in_specs=[
          pl.BlockSpec((K, _RB, 1024), lambda i: (0, i, 0)),
          pl.BlockSpec((_RB, 1024), lambda i: (i, 0)),
          pl.BlockSpec((NT, K + 1), lambda i: (0, 0)),
      ],
      out_specs=pl.BlockSpec((NT, _RB, 1024), lambda i: (0, i, 0)),
      out_shape=jax.ShapeDtypeStruct((NT, 3125, 1024), jnp.float32),
  )(monr, m0r, coeffs)


def kernel(edge_index, batch_indices, coeffs):
  eidx = edge_index.reshape(2, NCK, 128)
  deg = _deg_call(eidx)
  ab, bb, g0, m0 = _prep_call(deg, batch_indices)
  mon, _ = _prop_call(eidx, ab, bb, g0)
  monr = mon.reshape(K, NPAD * BT // 1024, 1024)
  m0r = m0.reshape(NPAD * BT // 1024, 1024)
  out = _tay_call(monr, m0r, coeffs)
  return out.reshape(NT, NN, BT)


# speedup vs baseline: 15.1985x; 15.1985x over previous
import jax
import jax.numpy as jnp
from jax import lax
from jax.experimental import pallas as pl
from jax.experimental.pallas import tpu as pltpu
from jax.experimental.pallas import tpu_sc as plsc

NN = 100000
NE = 1600000
BT = 32
HB = 16
K = 6
NT = 4

NPAD = 100352
RPT = NPAD // 16
SCH = 128
NCKP = 12544
SPT = NCKP // 16 // 8

_mesh = plsc.VectorSubcoreMesh(core_axis_name="c", subcore_axis_name="s")


def _deg_body(eidx, deg, dacc, ibuf, obuf, zbuf):
  c = lax.axis_index("c")
  s = lax.axis_index("s")

  @pl.loop(0, SCH // 16)
  def _(i):
    zbuf[pl.ds(i * 16, 16)] = jnp.zeros((16,), jnp.float32)

  @pl.loop(0, 8)
  def _(i):
    obuf[pl.ds(i * 16, 16)] = jnp.ones((16,), jnp.float32)

  @pl.loop(0, RPT // SCH)
  def _(i):
    pltpu.sync_copy(zbuf, dacc.at[pl.ds(s * RPT + i * SCH, SCH)])

  plsc.subcore_barrier()

  lo = s * SPT * 8

  @pl.loop(0, SPT)
  def _(g):
    pltpu.sync_copy(eidx.at[c, pl.ds(lo + g * 8, 8)], ibuf)

    @pl.loop(0, 8)
    def _(q):
      pltpu.sync_copy(obuf, dacc.at[ibuf.at[q]], add=True)

  plsc.subcore_barrier()

  @pl.loop(0, RPT // SCH)
  def _(i):
    r0 = s * RPT + i * SCH
    pltpu.sync_copy(dacc.at[pl.ds(r0, SCH)], deg.at[c, pl.ds(r0, SCH)])


_deg_call = pl.kernel(
    _deg_body,
    out_type=jax.ShapeDtypeStruct((2, NPAD), jnp.float32),
    mesh=_mesh,
    compiler_params=pltpu.CompilerParams(use_tc_tiling_on_sc=False),
    scratch_types=[
        pltpu.VMEM_SHARED((NPAD,), jnp.float32),
        pltpu.VMEM((8, 128), jnp.int32),
        pltpu.VMEM((128,), jnp.float32),
        pltpu.VMEM((SCH,), jnp.float32),
    ],
)


_NBB = 1024


def _prep_body(deg_ref, bi_ref, ab_ref, bb_ref, a_ref, m0_ref):
  i = pl.program_id(0)
  dsr = deg_ref[0, :]
  ddt = deg_ref[1, :]
  a = jnp.where(dsr > 0, lax.rsqrt(dsr), 0.0)
  b = jnp.where(ddt > 0, lax.rsqrt(ddt), 0.0)
  ab_ref[...] = a * b
  bb_ref[...] = b
  a_ref[...] = a
  ids = i * _NBB + lax.broadcasted_iota(jnp.int32, (_NBB,), 0)
  oh = ids[:, None] == bi_ref[...][None, :]
  m0_ref[...] = oh.astype(jnp.float32)


def _prep_call(deg, bi):
  return pl.pallas_call(
      _prep_body,
      grid=(NPAD // _NBB,),
      in_specs=[
          pl.BlockSpec((2, _NBB), lambda i: (0, i)),
          pl.BlockSpec((BT,), lambda i: (0,)),
      ],
      out_specs=[
          pl.BlockSpec((_NBB,), lambda i: (i,)),
          pl.BlockSpec((_NBB,), lambda i: (i,)),
          pl.BlockSpec((_NBB,), lambda i: (i,)),
          pl.BlockSpec((_NBB, BT), lambda i: (i, 0)),
      ],
      out_shape=[
          jax.ShapeDtypeStruct((NPAD,), jnp.float32),
          jax.ShapeDtypeStruct((NPAD,), jnp.float32),
          jax.ShapeDtypeStruct((NPAD,), jnp.float32),
          jax.ShapeDtypeStruct((NPAD, BT), jnp.float32),
      ],
  )(deg, bi)


def _prop_body(eidx, ab, bb, av, bi, mon, gw,
               acc, sib, dib, rbuf, tbuf, gbuf, mbuf, abuf, bbuf, zbuf,
               bibuf, afb, hbuf, gsem):
  c = lax.axis_index("c")
  s = lax.axis_index("s")
  goff = c * NPAD

  @pl.loop(0, SCH)
  def _(n):
    zbuf[n] = jnp.zeros((16,), jnp.float32)

  @pl.loop(0, RPT // SCH)
  def _(i):
    pltpu.sync_copy(zbuf, acc.at[pl.ds(s * RPT + i * SCH, SCH)])

  plsc.subcore_barrier()

  @pl.when(s == 0)
  def _():
    pltpu.sync_copy(bi.at[pl.ds(c * HB, HB)], bibuf)
    pltpu.sync_copy(av.at[bibuf], afb)
    afv = afb[...]
    lane = lax.iota(jnp.int32, 16)
    for j in range(HB):
      hbuf[j] = jnp.where(lane == j, afv[j], 0.0)
    pltpu.sync_copy(hbuf, acc.at[bibuf], add=True)

  plsc.subcore_barrier()

  @pl.loop(0, RPT // SCH)
  def _(i):
    r0 = s * RPT + i * SCH
    pltpu.sync_copy(acc.at[pl.ds(r0, SCH)], gw.at[pl.ds(goff + r0, SCH)])
    pltpu.sync_copy(zbuf, acc.at[pl.ds(r0, SCH)])

  plsc.subcore_barrier()

  lo = s * SPT * 8

  @pl.loop(0, K)
  def _(it):
    @pl.loop(0, SPT)
    def _(g):
      pltpu.sync_copy(eidx.at[0, pl.ds(lo + g * 8, 8)], sib)
      pltpu.sync_copy(eidx.at[1, pl.ds(lo + g * 8, 8)], dib)

      @pl.loop(0, 8)
      def _(q):
        for p in range(8):
          sib[q, pl.ds(p * 16, 16)] = sib[q, pl.ds(p * 16, 16)] + goff

      @pl.loop(0, 2)
      def _(h):
        @pl.loop(0, 4)
        def _(q):
          pltpu.async_copy(gw.at[sib.at[h * 4 + q]], rbuf.at[q], gsem)

        @pl.loop(0, 4)
        def _(q):
          pltpu.make_async_copy(gw.at[sib.at[h * 4 + q]], rbuf.at[q], gsem).wait()

        @pl.loop(0, 4)
        def _(q):
          pltpu.sync_copy(rbuf.at[q], acc.at[dib.at[h * 4 + q]], add=True)

    plsc.subcore_barrier()

    @pl.loop(0, RPT // SCH)
    def _(i):
      r0 = s * RPT + i * SCH
      pltpu.sync_copy(acc.at[pl.ds(r0, SCH)], tbuf)
      pltpu.sync_copy(ab.at[pl.ds(r0, SCH)], abuf)
      pltpu.sync_copy(bb.at[pl.ds(r0, SCH)], bbuf)
      pltpu.sync_copy(zbuf, acc.at[pl.ds(r0, SCH)])

      @pl.loop(0, SCH // 16)
      def _(g):
        avv = abuf[pl.ds(g * 16, 16)]
        bvv = bbuf[pl.ds(g * 16, 16)]
        for r in range(16):
          n = g * 16 + r
          t = tbuf[n]
          gbuf[n] = avv[r] * t
          mbuf[n] = bvv[r] * t

      pltpu.sync_copy(gbuf, gw.at[pl.ds(goff + r0, SCH)])
      pltpu.sync_copy(mbuf, mon.at[it, c, pl.ds(r0, SCH)])

    plsc.subcore_barrier()


_prop_call = pl.kernel(
    _prop_body,
    out_type=[
        jax.ShapeDtypeStruct((K, 2, NPAD, HB), jnp.float32),
        jax.ShapeDtypeStruct((2 * NPAD, HB), jnp.float32),
    ],
    mesh=_mesh,
    compiler_params=pltpu.CompilerParams(use_tc_tiling_on_sc=False),
    scratch_types=[
        pltpu.VMEM_SHARED((NPAD, HB), jnp.float32),
        pltpu.VMEM((8, 128), jnp.int32),
        pltpu.VMEM((8, 128), jnp.int32),
        pltpu.VMEM((4, 128, HB), jnp.float32),
        pltpu.VMEM((SCH, HB), jnp.float32),
        pltpu.VMEM((SCH, HB), jnp.float32),
        pltpu.VMEM((SCH, HB), jnp.float32),
        pltpu.VMEM((SCH,), jnp.float32),
        pltpu.VMEM((SCH,), jnp.float32),
        pltpu.VMEM((SCH, HB), jnp.float32),
        pltpu.VMEM((HB,), jnp.int32),
        pltpu.VMEM((HB,), jnp.float32),
        pltpu.VMEM((HB, HB), jnp.float32),
        pltpu.SemaphoreType.DMA,
    ],
)


_NR = 800


def _tay_body(mon_ref, m0_ref, co_ref, out_ref):
  co = co_ref[...]
  m0 = m0_ref[...]
  monh = mon_ref[...]
  for t in range(NT):
    accl = jnp.zeros((_NR, HB), jnp.float32)
    accr = jnp.zeros((_NR, HB), jnp.float32)
    for k in range(K):
      accl = accl + co[t, k + 1] * monh[k, 0]
      accr = accr + co[t, k + 1] * monh[k, 1]
    out_ref[t] = co[t, 0] * m0 + jnp.concatenate([accl, accr], axis=-1)


def _tay_call(mon, m0, coeffs):
  return pl.pallas_call(
      _tay_body,
      grid=(NN // _NR,),
      in_specs=[
          pl.BlockSpec((K, 2, _NR, HB), lambda i: (0, 0, i, 0)),
          pl.BlockSpec((_NR, BT), lambda i: (i, 0)),
          pl.BlockSpec((NT, K + 1), lambda i: (0, 0)),
      ],
      out_specs=pl.BlockSpec((NT, _NR, BT), lambda i: (0, i, 0)),
      out_shape=jax.ShapeDtypeStruct((NT, NN, BT), jnp.float32),
  )(mon, m0, coeffs)


def kernel(edge_index, batch_indices, coeffs):
  npad_e = NCKP * 128 - NE
  dead = NN + (jnp.arange(npad_e, dtype=jnp.int32) % (NPAD - NN))
  pad = jnp.stack([dead, dead])
  eidx = jnp.concatenate([edge_index, pad], axis=1).reshape(2, NCKP, 128)
  deg = _deg_call(eidx)
  ab, bb, av, m0 = _prep_call(deg, batch_indices)
  mon, _ = _prop_call(eidx, ab, bb, av, batch_indices)
  return _tay_call(mon, m0, coeffs)


# speedup vs baseline: 23.2677x; 1.5309x over previous
import jax
import jax.numpy as jnp
from jax import lax
from jax.experimental import pallas as pl
from jax.experimental.pallas import tpu as pltpu
from jax.experimental.pallas import tpu_sc as plsc

NN = 100000
NE = 1600000
BT = 32
HB = 16
K = 6
NT = 4

NPAD = 100352
RPT = NPAD // 16
SCH = 128
NCKP = 12544
SPT = NCKP // 16 // 8

_mesh = plsc.VectorSubcoreMesh(core_axis_name="c", subcore_axis_name="s")


def _deg_body(eidx, deg, dacc, ibuf, obuf, zbuf, isem):
  c = lax.axis_index("c")
  s = lax.axis_index("s")

  @pl.loop(0, SCH // 16)
  def _(i):
    zbuf[pl.ds(i * 16, 16)] = jnp.zeros((16,), jnp.float32)

  @pl.loop(0, 8)
  def _(i):
    obuf[pl.ds(i * 16, 16)] = jnp.ones((16,), jnp.float32)

  @pl.loop(0, RPT // SCH)
  def _(i):
    pltpu.sync_copy(zbuf, dacc.at[pl.ds(s * RPT + i * SCH, SCH)])

  plsc.subcore_barrier()

  lo = s * SPT * 8
  pltpu.sync_copy(eidx.at[c, pl.ds(lo, 8)], ibuf.at[0])

  @pl.loop(0, SPT)
  def _(g):
    p = lax.rem(g, 2)

    @pl.when(g > 0)
    def _():
      pltpu.make_async_copy(
          eidx.at[c, pl.ds(lo + g * 8, 8)], ibuf.at[p], isem).wait()

    @pl.when(g + 1 < SPT)
    def _():
      pltpu.async_copy(
          eidx.at[c, pl.ds(lo + (g + 1) * 8, 8)], ibuf.at[1 - p], isem)

    for q in range(8):
      pltpu.sync_copy(obuf, dacc.at[ibuf.at[p, q]], add=True)

  plsc.subcore_barrier()

  @pl.loop(0, RPT // SCH)
  def _(i):
    r0 = s * RPT + i * SCH
    pltpu.sync_copy(dacc.at[pl.ds(r0, SCH)], deg.at[c, pl.ds(r0, SCH)])


_deg_call = pl.kernel(
    _deg_body,
    out_type=jax.ShapeDtypeStruct((2, NPAD), jnp.float32),
    mesh=_mesh,
    compiler_params=pltpu.CompilerParams(use_tc_tiling_on_sc=False),
    scratch_types=[
        pltpu.VMEM_SHARED((NPAD,), jnp.float32),
        pltpu.VMEM((2, 8, 128), jnp.int32),
        pltpu.VMEM((128,), jnp.float32),
        pltpu.VMEM((SCH,), jnp.float32),
        pltpu.SemaphoreType.DMA,
    ],
)


_NBB = 1024


def _prep_body(deg_ref, bi_ref, ab_ref, bb_ref, a_ref, m0_ref):
  i = pl.program_id(0)
  dsr = deg_ref[0, :]
  ddt = deg_ref[1, :]
  a = jnp.where(dsr > 0, lax.rsqrt(dsr), 0.0)
  b = jnp.where(ddt > 0, lax.rsqrt(ddt), 0.0)
  ab_ref[...] = a * b
  bb_ref[...] = b
  a_ref[...] = a
  ids = i * _NBB + lax.broadcasted_iota(jnp.int32, (_NBB,), 0)
  oh = ids[:, None] == bi_ref[...][None, :]
  m0_ref[...] = oh.astype(jnp.float32)


def _prep_call(deg, bi):
  return pl.pallas_call(
      _prep_body,
      grid=(NPAD // _NBB,),
      in_specs=[
          pl.BlockSpec((2, _NBB), lambda i: (0, i)),
          pl.BlockSpec((BT,), lambda i: (0,)),
      ],
      out_specs=[
          pl.BlockSpec((_NBB,), lambda i: (i,)),
          pl.BlockSpec((_NBB,), lambda i: (i,)),
          pl.BlockSpec((_NBB,), lambda i: (i,)),
          pl.BlockSpec((_NBB, BT), lambda i: (i, 0)),
      ],
      out_shape=[
          jax.ShapeDtypeStruct((NPAD,), jnp.float32),
          jax.ShapeDtypeStruct((NPAD,), jnp.float32),
          jax.ShapeDtypeStruct((NPAD,), jnp.float32),
          jax.ShapeDtypeStruct((NPAD, BT), jnp.float32),
      ],
  )(deg, bi)


def _prop_body(eidx, ab, bb, av, bi, mon, gw,
               acc, sib, dib, rbuf, tbuf, gbuf, mbuf, abuf, bbuf, zbuf,
               bibuf, afb, hbuf, isem, gsem):
  c = lax.axis_index("c")
  s = lax.axis_index("s")
  goff = c * NPAD

  @pl.loop(0, SCH)
  def _(n):
    zbuf[n] = jnp.zeros((16,), jnp.float32)

  @pl.loop(0, RPT // SCH)
  def _(i):
    pltpu.sync_copy(zbuf, acc.at[pl.ds(s * RPT + i * SCH, SCH)])

  plsc.subcore_barrier()

  @pl.when(s == 0)
  def _():
    pltpu.sync_copy(bi.at[pl.ds(c * HB, HB)], bibuf)
    pltpu.sync_copy(av.at[bibuf], afb)
    afv = afb[...]
    lane = lax.iota(jnp.int32, 16)
    for j in range(HB):
      hbuf[j] = jnp.where(lane == j, afv[j], 0.0)
    pltpu.sync_copy(hbuf, acc.at[bibuf], add=True)

  plsc.subcore_barrier()

  @pl.loop(0, RPT // SCH)
  def _(i):
    r0 = s * RPT + i * SCH
    pltpu.sync_copy(acc.at[pl.ds(r0, SCH)], gw.at[pl.ds(goff + r0, SCH)])
    pltpu.sync_copy(zbuf, acc.at[pl.ds(r0, SCH)])

  plsc.subcore_barrier()

  lo = s * SPT * 8
  nsc = RPT // SCH

  @pl.loop(0, K)
  def _(it):
    pltpu.sync_copy(eidx.at[0, pl.ds(lo, 8)], sib.at[0])
    pltpu.sync_copy(eidx.at[1, pl.ds(lo, 8)], dib.at[0])

    @pl.loop(0, 8)
    def _(q):
      for pp in range(8):
        sib[0, q, pl.ds(pp * 16, 16)] = sib[0, q, pl.ds(pp * 16, 16)] + goff

    for q in range(4):
      pltpu.async_copy(gw.at[sib.at[0, q]], rbuf.at[0, q], gsem)

    @pl.loop(0, SPT)
    def _(g):
      p = lax.rem(g, 2)

      @pl.when(g + 1 < SPT)
      def _():
        pltpu.async_copy(
            eidx.at[0, pl.ds(lo + (g + 1) * 8, 8)], sib.at[1 - p], isem)
        pltpu.async_copy(
            eidx.at[1, pl.ds(lo + (g + 1) * 8, 8)], dib.at[1 - p], isem)

      for q in range(4):
        pltpu.async_copy(gw.at[sib.at[p, 4 + q]], rbuf.at[1, q], gsem)
      for q in range(4):
        pltpu.make_async_copy(gw.at[sib.at[p, q]], rbuf.at[0, q], gsem).wait()
        pltpu.sync_copy(rbuf.at[0, q], acc.at[dib.at[p, q]], add=True)

      @pl.when(g + 1 < SPT)
      def _():
        pltpu.make_async_copy(
            eidx.at[0, pl.ds(lo + (g + 1) * 8, 8)], sib.at[1 - p], isem).wait()
        pltpu.make_async_copy(
            eidx.at[1, pl.ds(lo + (g + 1) * 8, 8)], dib.at[1 - p], isem).wait()

        @pl.loop(0, 8)
        def _(q):
          for pp in range(8):
            sib[1 - p, q, pl.ds(pp * 16, 16)] = (
                sib[1 - p, q, pl.ds(pp * 16, 16)] + goff)

        for q in range(4):
          pltpu.async_copy(gw.at[sib.at[1 - p, q]], rbuf.at[0, q], gsem)

      for q in range(4):
        pltpu.make_async_copy(gw.at[sib.at[p, 4 + q]], rbuf.at[1, q], gsem).wait()
        pltpu.sync_copy(rbuf.at[1, q], acc.at[dib.at[p, 4 + q]], add=True)

    plsc.subcore_barrier()

    @pl.loop(0, nsc)
    def _(i):
      r0 = s * RPT + i * SCH
      pltpu.sync_copy(acc.at[pl.ds(r0, SCH)], tbuf)
      pltpu.sync_copy(ab.at[pl.ds(r0, SCH)], abuf)
      pltpu.sync_copy(bb.at[pl.ds(r0, SCH)], bbuf)
      pltpu.sync_copy(zbuf, acc.at[pl.ds(r0, SCH)])

      @pl.loop(0, SCH // 16)
      def _(g):
        avv = abuf[pl.ds(g * 16, 16)]
        bvv = bbuf[pl.ds(g * 16, 16)]
        for r in range(16):
          n = g * 16 + r
          t = tbuf[n]
          gbuf[n] = avv[r] * t
          mbuf[n] = bvv[r] * t

      pltpu.sync_copy(gbuf, gw.at[pl.ds(goff + r0, SCH)])
      pltpu.sync_copy(mbuf, mon.at[it, c, pl.ds(r0, SCH)])

    plsc.subcore_barrier()


_prop_call = pl.kernel(
    _prop_body,
    out_type=[
        jax.ShapeDtypeStruct((K, 2, NPAD, HB), jnp.float32),
        jax.ShapeDtypeStruct((2 * NPAD, HB), jnp.float32),
    ],
    mesh=_mesh,
    compiler_params=pltpu.CompilerParams(use_tc_tiling_on_sc=False),
    scratch_types=[
        pltpu.VMEM_SHARED((NPAD, HB), jnp.float32),
        pltpu.VMEM((2, 8, 128), jnp.int32),
        pltpu.VMEM((2, 8, 128), jnp.int32),
        pltpu.VMEM((2, 4, 128, HB), jnp.float32),
        pltpu.VMEM((SCH, HB), jnp.float32),
        pltpu.VMEM((SCH, HB), jnp.float32),
        pltpu.VMEM((SCH, HB), jnp.float32),
        pltpu.VMEM((SCH,), jnp.float32),
        pltpu.VMEM((SCH,), jnp.float32),
        pltpu.VMEM((SCH, HB), jnp.float32),
        pltpu.VMEM((HB,), jnp.int32),
        pltpu.VMEM((HB,), jnp.float32),
        pltpu.VMEM((HB, HB), jnp.float32),
        pltpu.SemaphoreType.DMA,
        pltpu.SemaphoreType.DMA,
    ],
)


_NR = 800


def _tay_body(mon_ref, m0_ref, co_ref, out_ref):
  co = co_ref[...]
  m0 = m0_ref[...]
  monh = mon_ref[...]
  for t in range(NT):
    accl = jnp.zeros((_NR, HB), jnp.float32)
    accr = jnp.zeros((_NR, HB), jnp.float32)
    for k in range(K):
      accl = accl + co[t, k + 1] * monh[k, 0]
      accr = accr + co[t, k + 1] * monh[k, 1]
    out_ref[t] = co[t, 0] * m0 + jnp.concatenate([accl, accr], axis=-1)


def _tay_call(mon, m0, coeffs):
  return pl.pallas_call(
      _tay_body,
      grid=(NN // _NR,),
      in_specs=[
          pl.BlockSpec((K, 2, _NR, HB), lambda i: (0, 0, i, 0)),
          pl.BlockSpec((_NR, BT), lambda i: (i, 0)),
          pl.BlockSpec((NT, K + 1), lambda i: (0, 0)),
      ],
      out_specs=pl.BlockSpec((NT, _NR, BT), lambda i: (0, i, 0)),
      out_shape=jax.ShapeDtypeStruct((NT, NN, BT), jnp.float32),
  )(mon, m0, coeffs)


def kernel(edge_index, batch_indices, coeffs):
  npad_e = NCKP * 128 - NE
  dead = NN + (jnp.arange(npad_e, dtype=jnp.int32) % (NPAD - NN))
  pad = jnp.stack([dead, dead])
  eidx = jnp.concatenate([edge_index, pad], axis=1).reshape(2, NCKP, 128)
  deg = _deg_call(eidx)
  ab, bb, av, m0 = _prep_call(deg, batch_indices)
  mon, _ = _prop_call(eidx, ab, bb, av, batch_indices)
  return _tay_call(mon, m0, coeffs)


# speedup vs baseline: 23.4695x; 1.0087x over previous
import jax
import jax.numpy as jnp
from jax import lax
from jax.experimental import pallas as pl
from jax.experimental.pallas import tpu as pltpu
from jax.experimental.pallas import tpu_sc as plsc

NN = 100000
NE = 1600000
BT = 32
HB = 16
K = 6
NT = 4

NPAD = 100352
RPT = NPAD // 16
SCH = 128
NCKP = 12544
SPT = NCKP // 16 // 8

_mesh = plsc.VectorSubcoreMesh(core_axis_name="c", subcore_axis_name="s")


def _deg_body(eidx, deg, dacc, ibuf, obuf, zbuf, isem):
  c = lax.axis_index("c")
  s = lax.axis_index("s")

  @pl.loop(0, SCH // 16)
  def _(i):
    zbuf[pl.ds(i * 16, 16)] = jnp.zeros((16,), jnp.float32)

  @pl.loop(0, 8)
  def _(i):
    obuf[pl.ds(i * 16, 16)] = jnp.ones((16,), jnp.float32)

  @pl.loop(0, RPT // SCH)
  def _(i):
    pltpu.sync_copy(zbuf, dacc.at[pl.ds(s * RPT + i * SCH, SCH)])

  plsc.subcore_barrier()

  lo = s * SPT * 8
  pltpu.sync_copy(eidx.at[c, pl.ds(lo, 8)], ibuf.at[0])

  @pl.loop(0, SPT)
  def _(g):
    p = lax.rem(g, 2)

    @pl.when(g > 0)
    def _():
      pltpu.make_async_copy(
          eidx.at[c, pl.ds(lo + g * 8, 8)], ibuf.at[p], isem).wait()

    @pl.when(g + 1 < SPT)
    def _():
      pltpu.async_copy(
          eidx.at[c, pl.ds(lo + (g + 1) * 8, 8)], ibuf.at[1 - p], isem)

    for q in range(8):
      pltpu.sync_copy(obuf, dacc.at[ibuf.at[p, q]], add=True)

  plsc.subcore_barrier()

  @pl.loop(0, RPT // SCH)
  def _(i):
    r0 = s * RPT + i * SCH
    pltpu.sync_copy(dacc.at[pl.ds(r0, SCH)], deg.at[c, pl.ds(r0, SCH)])


_deg_call = pl.kernel(
    _deg_body,
    out_type=jax.ShapeDtypeStruct((2, NPAD), jnp.float32),
    mesh=_mesh,
    compiler_params=pltpu.CompilerParams(use_tc_tiling_on_sc=False),
    scratch_types=[
        pltpu.VMEM_SHARED((NPAD,), jnp.float32),
        pltpu.VMEM((2, 8, 128), jnp.int32),
        pltpu.VMEM((128,), jnp.float32),
        pltpu.VMEM((SCH,), jnp.float32),
        pltpu.SemaphoreType.DMA,
    ],
)


_NBB = 1024


def _prep_body(deg_ref, bi_ref, ab_ref, bb_ref, a_ref, m0_ref):
  i = pl.program_id(0)
  dsr = deg_ref[0, :]
  ddt = deg_ref[1, :]
  a = jnp.where(dsr > 0, lax.rsqrt(dsr), 0.0)
  b = jnp.where(ddt > 0, lax.rsqrt(ddt), 0.0)
  ab_ref[...] = a * b
  bb_ref[...] = b
  a_ref[...] = a
  ids = i * _NBB + lax.broadcasted_iota(jnp.int32, (_NBB,), 0)
  oh = ids[:, None] == bi_ref[...][None, :]
  m0_ref[...] = oh.astype(jnp.float32)


def _prep_call(deg, bi):
  return pl.pallas_call(
      _prep_body,
      grid=(NPAD // _NBB,),
      in_specs=[
          pl.BlockSpec((2, _NBB), lambda i: (0, i)),
          pl.BlockSpec((BT,), lambda i: (0,)),
      ],
      out_specs=[
          pl.BlockSpec((_NBB,), lambda i: (i,)),
          pl.BlockSpec((_NBB,), lambda i: (i,)),
          pl.BlockSpec((_NBB,), lambda i: (i,)),
          pl.BlockSpec((_NBB, BT), lambda i: (i, 0)),
      ],
      out_shape=[
          jax.ShapeDtypeStruct((NPAD,), jnp.float32),
          jax.ShapeDtypeStruct((NPAD,), jnp.float32),
          jax.ShapeDtypeStruct((NPAD,), jnp.float32),
          jax.ShapeDtypeStruct((NPAD, BT), jnp.float32),
      ],
  )(deg, bi)


def _prop_body(eidx, ab, bb, av, bi, mon, gw,
               acc, sib, dib, rbuf, tbuf, gbuf, mbuf, abuf, bbuf, zbuf,
               bibuf, afb, hbuf, isem, gsem, ssem):
  c = lax.axis_index("c")
  s = lax.axis_index("s")
  goff = c * NPAD

  @pl.loop(0, SCH)
  def _(n):
    zbuf[n] = jnp.zeros((16,), jnp.float32)

  @pl.loop(0, RPT // SCH)
  def _(i):
    pltpu.sync_copy(zbuf, acc.at[pl.ds(s * RPT + i * SCH, SCH)])

  plsc.subcore_barrier()

  @pl.when(s == 0)
  def _():
    pltpu.sync_copy(bi.at[pl.ds(c * HB, HB)], bibuf)
    pltpu.sync_copy(av.at[bibuf], afb)
    afv = afb[...]
    lane = lax.iota(jnp.int32, 16)
    for j in range(HB):
      hbuf[j] = jnp.where(lane == j, afv[j], 0.0)
    pltpu.sync_copy(hbuf, acc.at[bibuf], add=True)

  plsc.subcore_barrier()

  @pl.loop(0, RPT // SCH)
  def _(i):
    r0 = s * RPT + i * SCH
    pltpu.sync_copy(acc.at[pl.ds(r0, SCH)], gw.at[pl.ds(goff + r0, SCH)])
    pltpu.sync_copy(zbuf, acc.at[pl.ds(r0, SCH)])

  plsc.subcore_barrier()

  lo = s * SPT * 8
  nsc = RPT // SCH

  @pl.loop(0, K)
  def _(it):
    pltpu.sync_copy(eidx.at[0, pl.ds(lo, 8)], sib.at[0])
    pltpu.sync_copy(eidx.at[1, pl.ds(lo, 8)], dib.at[0])

    @pl.loop(0, SPT)
    def _(g):
      p = lax.rem(g, 2)

      @pl.when(g > 0)
      def _():
        pltpu.make_async_copy(
            eidx.at[0, pl.ds(lo + g * 8, 8)], sib.at[p], isem).wait()
        pltpu.make_async_copy(
            eidx.at[1, pl.ds(lo + g * 8, 8)], dib.at[p], isem).wait()

      @pl.loop(0, 8)
      def _(q):
        for pp in range(8):
          sib[p, q, pl.ds(pp * 16, 16)] = sib[p, q, pl.ds(pp * 16, 16)] + goff

      for q in range(8):
        @pl.when(g > 0)
        def _():
          pltpu.make_async_copy(
              rbuf.at[q], acc.at[dib.at[1 - p, q]], ssem).wait()
        pltpu.async_copy(gw.at[sib.at[p, q]], rbuf.at[q], gsem)

      @pl.when(g + 1 < SPT)
      def _():
        pltpu.async_copy(
            eidx.at[0, pl.ds(lo + (g + 1) * 8, 8)], sib.at[1 - p], isem)
        pltpu.async_copy(
            eidx.at[1, pl.ds(lo + (g + 1) * 8, 8)], dib.at[1 - p], isem)

      for q in range(8):
        pltpu.make_async_copy(gw.at[sib.at[p, q]], rbuf.at[q], gsem).wait()
        pltpu.async_copy(rbuf.at[q], acc.at[dib.at[p, q]], ssem, add=True)

    @pl.loop(0, 8)
    def _(q):
      pltpu.make_async_copy(rbuf.at[q], acc.at[dib.at[1, q]], ssem).wait()

    plsc.subcore_barrier()

    @pl.loop(0, nsc)
    def _(i):
      r0 = s * RPT + i * SCH
      pltpu.sync_copy(acc.at[pl.ds(r0, SCH)], tbuf)
      pltpu.sync_copy(ab.at[pl.ds(r0, SCH)], abuf)
      pltpu.sync_copy(bb.at[pl.ds(r0, SCH)], bbuf)
      pltpu.sync_copy(zbuf, acc.at[pl.ds(r0, SCH)])

      @pl.loop(0, SCH // 16)
      def _(g):
        avv = abuf[pl.ds(g * 16, 16)]
        bvv = bbuf[pl.ds(g * 16, 16)]
        for r in range(16):
          n = g * 16 + r
          t = tbuf[n]
          gbuf[n] = avv[r] * t
          mbuf[n] = bvv[r] * t

      pltpu.sync_copy(gbuf, gw.at[pl.ds(goff + r0, SCH)])
      pltpu.sync_copy(mbuf, mon.at[it, c, pl.ds(r0, SCH)])

    plsc.subcore_barrier()


_prop_call = pl.kernel(
    _prop_body,
    out_type=[
        jax.ShapeDtypeStruct((K, 2, NPAD, HB), jnp.float32),
        jax.ShapeDtypeStruct((2 * NPAD, HB), jnp.float32),
    ],
    mesh=_mesh,
    compiler_params=pltpu.CompilerParams(use_tc_tiling_on_sc=False),
    scratch_types=[
        pltpu.VMEM_SHARED((NPAD, HB), jnp.float32),
        pltpu.VMEM((2, 8, 128), jnp.int32),
        pltpu.VMEM((2, 8, 128), jnp.int32),
        pltpu.VMEM((8, 128, HB), jnp.float32),
        pltpu.VMEM((SCH, HB), jnp.float32),
        pltpu.VMEM((SCH, HB), jnp.float32),
        pltpu.VMEM((SCH, HB), jnp.float32),
        pltpu.VMEM((SCH,), jnp.float32),
        pltpu.VMEM((SCH,), jnp.float32),
        pltpu.VMEM((SCH, HB), jnp.float32),
        pltpu.VMEM((HB,), jnp.int32),
        pltpu.VMEM((HB,), jnp.float32),
        pltpu.VMEM((HB, HB), jnp.float32),
        pltpu.SemaphoreType.DMA,
        pltpu.SemaphoreType.DMA,
        pltpu.SemaphoreType.DMA,
    ],
)


_NR = 800


def _tay_body(mon_ref, m0_ref, co_ref, out_ref):
  co = co_ref[...]
  m0 = m0_ref[...]
  monh = mon_ref[...]
  for t in range(NT):
    accl = jnp.zeros((_NR, HB), jnp.float32)
    accr = jnp.zeros((_NR, HB), jnp.float32)
    for k in range(K):
      accl = accl + co[t, k + 1] * monh[k, 0]
      accr = accr + co[t, k + 1] * monh[k, 1]
    out_ref[t] = co[t, 0] * m0 + jnp.concatenate([accl, accr], axis=-1)


def _tay_call(mon, m0, coeffs):
  return pl.pallas_call(
      _tay_body,
      grid=(NN // _NR,),
      in_specs=[
          pl.BlockSpec((K, 2, _NR, HB), lambda i: (0, 0, i, 0)),
          pl.BlockSpec((_NR, BT), lambda i: (i, 0)),
          pl.BlockSpec((NT, K + 1), lambda i: (0, 0)),
      ],
      out_specs=pl.BlockSpec((NT, _NR, BT), lambda i: (0, i, 0)),
      out_shape=jax.ShapeDtypeStruct((NT, NN, BT), jnp.float32),
  )(mon, m0, coeffs)


def kernel(edge_index, batch_indices, coeffs):
  npad_e = NCKP * 128 - NE
  dead = NN + (jnp.arange(npad_e, dtype=jnp.int32) % (NPAD - NN))
  pad = jnp.stack([dead, dead])
  eidx = jnp.concatenate([edge_index, pad], axis=1).reshape(2, NCKP, 128)
  deg = _deg_call(eidx)
  ab, bb, av, m0 = _prep_call(deg, batch_indices)
  mon, _ = _prop_call(eidx, ab, bb, av, batch_indices)
  return _tay_call(mon, m0, coeffs)


# speedup vs baseline: 25.0100x; 1.0656x over previous
import jax
import jax.numpy as jnp
from jax import lax
from jax.experimental import pallas as pl
from jax.experimental.pallas import tpu as pltpu
from jax.experimental.pallas import tpu_sc as plsc

NN = 100000
NE = 1600000
BT = 32
HB = 16
K = 6
NT = 4

NPAD = 100352
RPT = NPAD // 16
SCH = 112
NCKP = 12544
SPT = NCKP // 16 // 8

_mesh = plsc.VectorSubcoreMesh(core_axis_name="c", subcore_axis_name="s")


def _deg_body(eidx, deg, dacc, ibuf, obuf, zbuf, isem, dsem):
  c = lax.axis_index("c")
  s = lax.axis_index("s")

  @pl.loop(0, SCH // 16)
  def _(i):
    zbuf[pl.ds(i * 16, 16)] = jnp.zeros((16,), jnp.float32)

  @pl.loop(0, 8)
  def _(i):
    obuf[pl.ds(i * 16, 16)] = jnp.ones((16,), jnp.float32)

  @pl.loop(0, RPT // SCH)
  def _(i):
    pltpu.sync_copy(zbuf, dacc.at[pl.ds(s * RPT + i * SCH, SCH)])

  plsc.subcore_barrier()

  lo = s * SPT * 8
  pltpu.sync_copy(eidx.at[c, pl.ds(lo, 8)], ibuf.at[0])

  @pl.loop(0, SPT)
  def _(g):
    p = lax.rem(g, 2)

    @pl.when(g > 0)
    def _():
      pltpu.make_async_copy(
          eidx.at[c, pl.ds(lo + g * 8, 8)], ibuf.at[p], isem).wait()

    @pl.when(g + 1 < SPT)
    def _():
      pltpu.async_copy(
          eidx.at[c, pl.ds(lo + (g + 1) * 8, 8)], ibuf.at[1 - p], isem)

    for q in range(8):
      @pl.when(g > 0)
      def _():
        pltpu.make_async_copy(obuf, dacc.at[ibuf.at[1 - p, q]], dsem).wait()
      pltpu.async_copy(obuf, dacc.at[ibuf.at[p, q]], dsem, add=True)

  @pl.loop(0, 8)
  def _(q):
    pltpu.make_async_copy(obuf, dacc.at[ibuf.at[1, q]], dsem).wait()

  plsc.subcore_barrier()

  @pl.loop(0, RPT // SCH)
  def _(i):
    r0 = s * RPT + i * SCH
    pltpu.sync_copy(dacc.at[pl.ds(r0, SCH)], deg.at[c, pl.ds(r0, SCH)])


_deg_call = pl.kernel(
    _deg_body,
    out_type=jax.ShapeDtypeStruct((2, NPAD), jnp.float32),
    mesh=_mesh,
    compiler_params=pltpu.CompilerParams(use_tc_tiling_on_sc=False),
    scratch_types=[
        pltpu.VMEM_SHARED((NPAD,), jnp.float32),
        pltpu.VMEM((2, 8, 128), jnp.int32),
        pltpu.VMEM((128,), jnp.float32),
        pltpu.VMEM((SCH,), jnp.float32),
        pltpu.SemaphoreType.DMA,
        pltpu.SemaphoreType.DMA,
    ],
)


_NBB = 1024


def _prep_body(deg_ref, bi_ref, ab_ref, bb_ref, a_ref, m0_ref):
  i = pl.program_id(0)
  dsr = deg_ref[0, :]
  ddt = deg_ref[1, :]
  a = jnp.where(dsr > 0, lax.rsqrt(dsr), 0.0)
  b = jnp.where(ddt > 0, lax.rsqrt(ddt), 0.0)
  ab_ref[...] = a * b
  bb_ref[...] = b
  a_ref[...] = a
  ids = i * _NBB + lax.broadcasted_iota(jnp.int32, (_NBB,), 0)
  oh = ids[:, None] == bi_ref[...][None, :]
  m0_ref[...] = oh.astype(jnp.float32)


def _prep_call(deg, bi):
  return pl.pallas_call(
      _prep_body,
      grid=(NPAD // _NBB,),
      in_specs=[
          pl.BlockSpec((2, _NBB), lambda i: (0, i)),
          pl.BlockSpec((BT,), lambda i: (0,)),
      ],
      out_specs=[
          pl.BlockSpec((_NBB,), lambda i: (i,)),
          pl.BlockSpec((_NBB,), lambda i: (i,)),
          pl.BlockSpec((_NBB,), lambda i: (i,)),
          pl.BlockSpec((_NBB, BT), lambda i: (i, 0)),
      ],
      out_shape=[
          jax.ShapeDtypeStruct((NPAD,), jnp.float32),
          jax.ShapeDtypeStruct((NPAD,), jnp.float32),
          jax.ShapeDtypeStruct((NPAD,), jnp.float32),
          jax.ShapeDtypeStruct((NPAD, BT), jnp.float32),
      ],
  )(deg, bi)


def _prop_body(eidx, ab, bb, av, bi, mon, gw,
               acc, sib, dib, rbuf, tbuf, gbuf, mbuf, abuf, bbuf, zbuf,
               bibuf, afb, hbuf, isem, gsem, ssem, rsemt, rsema, rsemb):
  c = lax.axis_index("c")
  s = lax.axis_index("s")
  goff = c * NPAD

  @pl.loop(0, SCH)
  def _(n):
    zbuf[n] = jnp.zeros((16,), jnp.float32)

  @pl.loop(0, RPT // SCH)
  def _(i):
    pltpu.sync_copy(zbuf, acc.at[pl.ds(s * RPT + i * SCH, SCH)])

  plsc.subcore_barrier()

  @pl.when(s == 0)
  def _():
    pltpu.sync_copy(bi.at[pl.ds(c * HB, HB)], bibuf)
    pltpu.sync_copy(av.at[bibuf], afb)
    afv = afb[...]
    lane = lax.iota(jnp.int32, 16)
    for j in range(HB):
      hbuf[j] = jnp.where(lane == j, afv[j], 0.0)
    pltpu.sync_copy(hbuf, acc.at[bibuf], add=True)

  plsc.subcore_barrier()

  @pl.loop(0, RPT // SCH)
  def _(i):
    r0 = s * RPT + i * SCH
    pltpu.sync_copy(acc.at[pl.ds(r0, SCH)], gw.at[pl.ds(goff + r0, SCH)])
    pltpu.sync_copy(zbuf, acc.at[pl.ds(r0, SCH)])

  plsc.subcore_barrier()

  lo = s * SPT * 8
  nsc = RPT // SCH

  @pl.loop(0, K)
  def _(it):
    pltpu.sync_copy(eidx.at[0, pl.ds(lo, 8)], sib.at[0])
    pltpu.sync_copy(eidx.at[1, pl.ds(lo, 8)], dib.at[0])

    @pl.loop(0, SPT)
    def _(g):
      p = lax.rem(g, 2)

      @pl.when(g > 0)
      def _():
        pltpu.make_async_copy(
            eidx.at[0, pl.ds(lo + g * 8, 8)], sib.at[p], isem).wait()
        pltpu.make_async_copy(
            eidx.at[1, pl.ds(lo + g * 8, 8)], dib.at[p], isem).wait()

      @pl.loop(0, 8)
      def _(q):
        for pp in range(8):
          sib[p, q, pl.ds(pp * 16, 16)] = sib[p, q, pl.ds(pp * 16, 16)] + goff

      for q in range(8):
        @pl.when(g > 0)
        def _():
          pltpu.make_async_copy(
              rbuf.at[q], acc.at[dib.at[1 - p, q]], ssem).wait()
        pltpu.async_copy(gw.at[sib.at[p, q]], rbuf.at[q], gsem)

      @pl.when(g + 1 < SPT)
      def _():
        pltpu.async_copy(
            eidx.at[0, pl.ds(lo + (g + 1) * 8, 8)], sib.at[1 - p], isem)
        pltpu.async_copy(
            eidx.at[1, pl.ds(lo + (g + 1) * 8, 8)], dib.at[1 - p], isem)

      for q in range(8):
        pltpu.make_async_copy(gw.at[sib.at[p, q]], rbuf.at[q], gsem).wait()
        pltpu.async_copy(rbuf.at[q], acc.at[dib.at[p, q]], ssem, add=True)

    @pl.loop(0, 8)
    def _(q):
      pltpu.make_async_copy(rbuf.at[q], acc.at[dib.at[1, q]], ssem).wait()

    plsc.subcore_barrier()

    r00 = s * RPT
    pltpu.async_copy(acc.at[pl.ds(r00, SCH)], tbuf.at[0], rsemt)
    pltpu.async_copy(ab.at[pl.ds(r00, SCH)], abuf.at[0], rsema)
    pltpu.async_copy(bb.at[pl.ds(r00, SCH)], bbuf.at[0], rsemb)

    @pl.loop(0, nsc)
    def _(i):
      p = lax.rem(i, 2)
      r0 = s * RPT + i * SCH
      pltpu.make_async_copy(acc.at[pl.ds(r0, SCH)], tbuf.at[p], rsemt).wait()
      pltpu.make_async_copy(ab.at[pl.ds(r0, SCH)], abuf.at[p], rsema).wait()
      pltpu.make_async_copy(bb.at[pl.ds(r0, SCH)], bbuf.at[p], rsemb).wait()
      pltpu.sync_copy(zbuf, acc.at[pl.ds(r0, SCH)])

      @pl.when(i + 1 < nsc)
      def _():
        r1 = r0 + SCH
        pltpu.async_copy(acc.at[pl.ds(r1, SCH)], tbuf.at[1 - p], rsemt)
        pltpu.async_copy(ab.at[pl.ds(r1, SCH)], abuf.at[1 - p], rsema)
        pltpu.async_copy(bb.at[pl.ds(r1, SCH)], bbuf.at[1 - p], rsemb)

      @pl.loop(0, SCH // 16)
      def _(g):
        avv = abuf[p, pl.ds(g * 16, 16)]
        bvv = bbuf[p, pl.ds(g * 16, 16)]
        for r in range(16):
          n = g * 16 + r
          t = tbuf[p, n]
          gbuf[n] = avv[r] * t
          mbuf[n] = bvv[r] * t

      pltpu.sync_copy(gbuf, gw.at[pl.ds(goff + r0, SCH)])
      pltpu.sync_copy(mbuf, mon.at[it, c, pl.ds(r0, SCH)])

    plsc.subcore_barrier()


_prop_call = pl.kernel(
    _prop_body,
    out_type=[
        jax.ShapeDtypeStruct((K, 2, NPAD, HB), jnp.float32),
        jax.ShapeDtypeStruct((2 * NPAD, HB), jnp.float32),
    ],
    mesh=_mesh,
    compiler_params=pltpu.CompilerParams(use_tc_tiling_on_sc=False),
    scratch_types=[
        pltpu.VMEM_SHARED((NPAD, HB), jnp.float32),
        pltpu.VMEM((2, 8, 128), jnp.int32),
        pltpu.VMEM((2, 8, 128), jnp.int32),
        pltpu.VMEM((8, 128, HB), jnp.float32),
        pltpu.VMEM((2, SCH, HB), jnp.float32),
        pltpu.VMEM((SCH, HB), jnp.float32),
        pltpu.VMEM((SCH, HB), jnp.float32),
        pltpu.VMEM((2, SCH), jnp.float32),
        pltpu.VMEM((2, SCH), jnp.float32),
        pltpu.VMEM((SCH, HB), jnp.float32),
        pltpu.VMEM((HB,), jnp.int32),
        pltpu.VMEM((HB,), jnp.float32),
        pltpu.VMEM((HB, HB), jnp.float32),
        pltpu.SemaphoreType.DMA,
        pltpu.SemaphoreType.DMA,
        pltpu.SemaphoreType.DMA,
        pltpu.SemaphoreType.DMA,
        pltpu.SemaphoreType.DMA,
        pltpu.SemaphoreType.DMA,
    ],
)


_NR = 800


def _tay_body(mon_ref, m0_ref, co_ref, out_ref):
  co = co_ref[...]
  m0 = m0_ref[...]
  monh = mon_ref[...]
  for t in range(NT):
    accl = jnp.zeros((_NR, HB), jnp.float32)
    accr = jnp.zeros((_NR, HB), jnp.float32)
    for k in range(K):
      accl = accl + co[t, k + 1] * monh[k, 0]
      accr = accr + co[t, k + 1] * monh[k, 1]
    out_ref[t] = co[t, 0] * m0 + jnp.concatenate([accl, accr], axis=-1)


def _tay_call(mon, m0, coeffs):
  return pl.pallas_call(
      _tay_body,
      grid=(NN // _NR,),
      in_specs=[
          pl.BlockSpec((K, 2, _NR, HB), lambda i: (0, 0, i, 0)),
          pl.BlockSpec((_NR, BT), lambda i: (i, 0)),
          pl.BlockSpec((NT, K + 1), lambda i: (0, 0)),
      ],
      out_specs=pl.BlockSpec((NT, _NR, BT), lambda i: (0, i, 0)),
      out_shape=jax.ShapeDtypeStruct((NT, NN, BT), jnp.float32),
  )(mon, m0, coeffs)


def kernel(edge_index, batch_indices, coeffs):
  npad_e = NCKP * 128 - NE
  dead = NN + (jnp.arange(npad_e, dtype=jnp.int32) % (NPAD - NN))
  pad = jnp.stack([dead, dead])
  eidx = jnp.concatenate([edge_index, pad], axis=1).reshape(2, NCKP, 128)
  deg = _deg_call(eidx)
  ab, bb, av, m0 = _prep_call(deg, batch_indices)
  mon, _ = _prop_call(eidx, ab, bb, av, batch_indices)
  return _tay_call(mon, m0, coeffs)
